# baseline (speedup 1.0000x reference)
import functools

import jax
import jax.numpy as jnp
from jax import lax
from jax.experimental import pallas as pl
from jax.experimental.pallas import tpu as pltpu
from jax.experimental.pallas import tpu_sc as plsc

VOCAB = 100000
EMB = 64
N_CTX = 16384

NW = 32
PER_W = N_CTX // NW
CHUNK = 128
NCHUNK = PER_W // CHUNK

BV = 2000
NBLK = VOCAB // BV


def _gather_sum_body(idx_hbm, table_hbm, out_hbm, idx_v, rows_v, acc_v, sem):
    wid = lax.axis_index("s") * 2 + lax.axis_index("c")
    pltpu.sync_copy(idx_hbm.at[wid], idx_v)
    copies = []
    for k in range(NCHUNK):
        copies.append(
            pltpu.async_copy(
                table_hbm.at[idx_v.at[k]],
                rows_v.at[pl.ds(k * CHUNK, CHUNK)],
                sem,
            )
        )
    for c in copies:
        c.wait()

    def body(r, carry):
        a0, a1, a2, a3 = carry
        a0 = a0 + rows_v[r, pl.ds(0, 16)]
        a1 = a1 + rows_v[r, pl.ds(16, 16)]
        a2 = a2 + rows_v[r, pl.ds(32, 16)]
        a3 = a3 + rows_v[r, pl.ds(48, 16)]
        return (a0, a1, a2, a3)

    z = jnp.zeros((16,), jnp.float32)
    a0, a1, a2, a3 = lax.fori_loop(0, PER_W, body, (z, z, z, z))
    acc_v[pl.ds(0, 16)] = a0
    acc_v[pl.ds(16, 16)] = a1
    acc_v[pl.ds(32, 16)] = a2
    acc_v[pl.ds(48, 16)] = a3
    pltpu.sync_copy(acc_v, out_hbm.at[wid])


@functools.cache
def _gather_sum():
    return functools.partial(
        pl.kernel,
        out_type=jax.ShapeDtypeStruct((NW, EMB), jnp.float32),
        mesh=plsc.VectorSubcoreMesh(core_axis_name="c", subcore_axis_name="s"),
        compiler_params=pltpu.CompilerParams(use_tc_tiling_on_sc=False),
        scratch_types=[
            pltpu.VMEM((NCHUNK, CHUNK), jnp.int32),
            pltpu.VMEM((PER_W, EMB), jnp.float32),
            pltpu.VMEM((EMB,), jnp.float32),
            pltpu.SemaphoreType.DMA,
        ],
    )(_gather_sum_body)


def _logits_body(partials_ref, b_ref, w_ref, logits_ref, bmax_ref, bsum_ref):
    s = jnp.sum(partials_ref[...], axis=0) * (1.0 / N_CTX)
    s2 = s.reshape(1, EMB)
    l = lax.dot_general(
        s2,
        w_ref[...],
        (((1,), (1,)), ((), ())),
        preferred_element_type=jnp.float32,
    )
    l = l.reshape(1, 1, BV) + b_ref[...]
    logits_ref[...] = l
    m = jnp.max(l)
    ones = jnp.ones((1, 1, 128), jnp.float32)
    bmax_ref[...] = m * ones
    bsum_ref[...] = jnp.sum(jnp.exp(l - m)) * ones


def _logits_call(partials, b_r, W):
    return pl.pallas_call(
        _logits_body,
        grid=(NBLK,),
        in_specs=[
            pl.BlockSpec((NW, EMB), lambda i: (0, 0)),
            pl.BlockSpec((1, 1, BV), lambda i: (i, 0, 0)),
            pl.BlockSpec((BV, EMB), lambda i: (i, 0)),
        ],
        out_specs=[
            pl.BlockSpec((1, 1, BV), lambda i: (i, 0, 0)),
            pl.BlockSpec((1, 1, 128), lambda i: (i, 0, 0)),
            pl.BlockSpec((1, 1, 128), lambda i: (i, 0, 0)),
        ],
        out_shape=[
            jax.ShapeDtypeStruct((NBLK, 1, BV), jnp.float32),
            jax.ShapeDtypeStruct((NBLK, 1, 128), jnp.float32),
            jax.ShapeDtypeStruct((NBLK, 1, 128), jnp.float32),
        ],
    )(partials, b_r, W)


def _finalize_body(bmax_ref, bsum_ref, logits_ref, out_ref):
    bm = bmax_ref[...]
    m = jnp.max(bm)
    S = jnp.sum(bsum_ref[...] * jnp.exp(bm - m)) * (1.0 / 128.0)
    out_ref[...] = logits_ref[...] - (m + jnp.log(S))


def _finalize_call(bmax, bsum, logits):
    return pl.pallas_call(
        _finalize_body,
        grid=(NBLK,),
        in_specs=[
            pl.BlockSpec((NBLK, 1, 128), lambda i: (0, 0, 0)),
            pl.BlockSpec((NBLK, 1, 128), lambda i: (0, 0, 0)),
            pl.BlockSpec((1, 1, BV), lambda i: (i, 0, 0)),
        ],
        out_specs=pl.BlockSpec((1, 1, BV), lambda i: (i, 0, 0)),
        out_shape=jax.ShapeDtypeStruct((NBLK, 1, BV), jnp.float32),
    )(bmax, bsum, logits)


def kernel(indices, emb_table, W, b):
    idx3 = indices.astype(jnp.int32).reshape(NW, NCHUNK, CHUNK)
    partials = _gather_sum()(idx3, emb_table)
    b_r = b.reshape(NBLK, 1, BV)
    logits, bmax, bsum = _logits_call(partials, b_r, W)
    out_r = _finalize_call(bmax, bsum, logits)
    return out_r.reshape(1, VOCAB)


# speedup vs baseline: 1.1057x; 1.1057x over previous
import functools

import jax
import jax.numpy as jnp
from jax import lax
from jax.experimental import pallas as pl
from jax.experimental.pallas import tpu as pltpu
from jax.experimental.pallas import tpu_sc as plsc

VOCAB = 100000
EMB = 64
N_CTX = 16384

NW = 32
PER_W = N_CTX // NW
CHUNK = 128
NCHUNK = PER_W // CHUNK

BV = 2048
NBLK = -(-VOCAB // BV)


def _gather_sum_body(idx_hbm, table_hbm, out_hbm, idx_v, rows_v, acc_v, sem):
    wid = lax.axis_index("s") * 2 + lax.axis_index("c")
    pltpu.sync_copy(idx_hbm.at[wid], idx_v)
    copies = []
    for k in range(NCHUNK):
        copies.append(
            pltpu.async_copy(
                table_hbm.at[idx_v.at[k]],
                rows_v.at[pl.ds(k * CHUNK, CHUNK)],
                sem,
            )
        )
    for c in copies:
        c.wait()

    def body(r, carry):
        a0, a1, a2, a3 = carry
        a0 = a0 + rows_v[r, pl.ds(0, 16)]
        a1 = a1 + rows_v[r, pl.ds(16, 16)]
        a2 = a2 + rows_v[r, pl.ds(32, 16)]
        a3 = a3 + rows_v[r, pl.ds(48, 16)]
        return (a0, a1, a2, a3)

    z = jnp.zeros((16,), jnp.float32)
    a0, a1, a2, a3 = lax.fori_loop(0, PER_W, body, (z, z, z, z))
    acc_v[pl.ds(0, 16)] = a0
    acc_v[pl.ds(16, 16)] = a1
    acc_v[pl.ds(32, 16)] = a2
    acc_v[pl.ds(48, 16)] = a3
    pltpu.sync_copy(acc_v, out_hbm.at[wid])


@functools.cache
def _gather_sum():
    return functools.partial(
        pl.kernel,
        out_type=jax.ShapeDtypeStruct((NW, EMB), jnp.float32),
        mesh=plsc.VectorSubcoreMesh(core_axis_name="c", subcore_axis_name="s"),
        compiler_params=pltpu.CompilerParams(use_tc_tiling_on_sc=False),
        scratch_types=[
            pltpu.VMEM((NCHUNK, CHUNK), jnp.int32),
            pltpu.VMEM((PER_W, EMB), jnp.float32),
            pltpu.VMEM((EMB,), jnp.float32),
            pltpu.SemaphoreType.DMA,
        ],
    )(_gather_sum_body)


def _tc_body(partials_ref, b_ref, w_ref, out_ref, logits_s, sm):
    p = pl.program_id(0)
    i = pl.program_id(1)

    @pl.when((p == 0) & (i == 0))
    def _init():
        sm[0] = -1e30
        sm[1] = 0.0

    @pl.when(p == 0)
    def _phase0():
        s2 = jnp.sum(partials_ref[...], axis=0, keepdims=True) * (1.0 / N_CTX)
        l = lax.dot_general(
            s2,
            w_ref[...],
            (((1,), (1,)), ((), ())),
            preferred_element_type=jnp.float32,
        ) + b_ref[...]
        lane = lax.broadcasted_iota(jnp.int32, (1, BV), 1) + i * BV
        l = jnp.where(lane < VOCAB, l, -1e30)
        logits_s[i] = l
        m_old = sm[0]
        m_new = jnp.maximum(m_old, jnp.max(l))
        sm[1] = sm[1] * jnp.exp(m_old - m_new) + jnp.sum(jnp.exp(l - m_new))
        sm[0] = m_new

    @pl.when(p == 1)
    def _phase1():
        out_ref[...] = logits_s[i] - (sm[0] + jnp.log(sm[1]))


def _tc_call(partials, b2, W):
    return pl.pallas_call(
        _tc_body,
        grid=(2, NBLK),
        in_specs=[
            pl.BlockSpec((NW, EMB), lambda p, i: (0, 0)),
            pl.BlockSpec((1, BV), lambda p, i: (0, jnp.where(p == 0, i, NBLK - 1))),
            pl.BlockSpec((BV, EMB), lambda p, i: (jnp.where(p == 0, i, NBLK - 1), 0)),
        ],
        out_specs=pl.BlockSpec((1, BV), lambda p, i: (0, jnp.where(p == 0, 0, i))),
        out_shape=jax.ShapeDtypeStruct((1, VOCAB), jnp.float32),
        scratch_shapes=[
            pltpu.VMEM((NBLK, 1, BV), jnp.float32),
            pltpu.SMEM((2,), jnp.float32),
        ],
    )(partials, b2, W)


def kernel(indices, emb_table, W, b):
    idx3 = indices.astype(jnp.int32).reshape(NW, NCHUNK, CHUNK)
    partials = _gather_sum()(idx3, emb_table)
    return _tc_call(partials, b.reshape(1, VOCAB), W)


# speedup vs baseline: 3.2205x; 2.9128x over previous
import functools

import jax
import jax.numpy as jnp
from jax import lax
from jax.experimental import pallas as pl
from jax.experimental.pallas import tpu as pltpu
from jax.experimental.pallas import tpu_sc as plsc

VOCAB = 100000
EMB = 64
N_CTX = 16384

NW = 32
PER_W = N_CTX // NW
CHUNK = 128
NCHUNK = PER_W // CHUNK

VOCAB_PAD = 100352
STRIPE = VOCAB_PAD // 16

BV = 7168
NBLK = VOCAB_PAD // BV


def _hist_body(idx_hbm, out_hbm, idx_v, zbuf, ones_v, shared, sem):
    core = lax.axis_index("c")
    sub = lax.axis_index("s")
    wid = sub * 2 + core

    z16 = jnp.zeros((16,), jnp.float32)

    def zero_body(j, _):
        zbuf[pl.ds(j * 16, 16)] = z16
        return 0

    lax.fori_loop(0, STRIPE // 16, zero_body, 0)
    for j in range(CHUNK // 16):
        ones_v[pl.ds(j * 16, 16)] = z16 + 1.0
    for k in range(NCHUNK):
        pltpu.sync_copy(
            idx_hbm.at[pl.ds(wid * PER_W + k * CHUNK, CHUNK)], idx_v.at[k]
        )
    pltpu.sync_copy(zbuf, shared.at[pl.ds(sub * STRIPE, STRIPE)])
    plsc.subcore_barrier()
    for k in range(NCHUNK):
        pltpu.sync_copy(ones_v, shared.at[idx_v.at[k]], add=True)
    plsc.subcore_barrier()
    pltpu.sync_copy(
        shared.at[pl.ds(sub * STRIPE, STRIPE)],
        out_hbm.at[pl.ds(core * VOCAB_PAD + sub * STRIPE, STRIPE)],
    )


@functools.cache
def _hist():
    return functools.partial(
        pl.kernel,
        out_type=jax.ShapeDtypeStruct((2 * VOCAB_PAD,), jnp.float32),
        mesh=plsc.VectorSubcoreMesh(core_axis_name="c", subcore_axis_name="s"),
        scratch_types=[
            pltpu.VMEM((NCHUNK, CHUNK), jnp.int32),
            pltpu.VMEM((STRIPE,), jnp.float32),
            pltpu.VMEM((CHUNK,), jnp.float32),
            pltpu.VMEM_SHARED((VOCAB_PAD,), jnp.float32),
            pltpu.SemaphoreType.DMA,
        ],
    )(_hist_body)


def _tc_body(counts_ref, t_ref, w_ref, b_ref, out_ref, s_acc, mv, sv, logits_s, sm):
    p = pl.program_id(0)
    i = pl.program_id(1)
    lane = lax.broadcasted_iota(jnp.int32, (1, BV), 1)
    nvalid = VOCAB - i * BV

    @pl.when((p == 0) & (i == 0))
    def _init_s():
        s_acc[...] = jnp.zeros((1, EMB), jnp.float32)

    @pl.when((p == 0) & (i < NBLK - 1))
    def _phase0():
        c2 = counts_ref[...]
        c = c2[0:1, :] + c2[1:2, :]
        s_acc[...] += lax.dot_general(
            c, t_ref[...], (((1,), (1,)), ((), ())),
            preferred_element_type=jnp.float32,
        )

    @pl.when((p == 0) & (i == NBLK - 1))
    def _phase0_edge():
        c2 = counts_ref[...]
        c = c2[0:1, :] + c2[1:2, :]
        t = jnp.where(lane < nvalid, t_ref[...], 0.0)
        s_acc[...] += lax.dot_general(
            c, t, (((1,), (1,)), ((), ())),
            preferred_element_type=jnp.float32,
        )

    @pl.when((p == 1) & (i == 0))
    def _init_ms():
        mv[...] = jnp.full((1, BV), -1e30, jnp.float32)
        sv[...] = jnp.zeros((1, BV), jnp.float32)

    @pl.when(p == 1)
    def _phase1():
        s2 = s_acc[...] * (1.0 / N_CTX)
        l = lax.dot_general(
            s2, w_ref[...], (((1,), (0,)), ((), ())),
            preferred_element_type=jnp.float32,
        ) + b_ref[...]
        l = jnp.where(lane < nvalid, l, -1e30)
        logits_s[i] = l
        m_old = mv[...]
        m_new = jnp.maximum(m_old, l)
        sv[...] = sv[...] * jnp.exp(m_old - m_new) + jnp.exp(l - m_new)
        mv[...] = m_new

    @pl.when((p == 2) & (i == 0))
    def _logz():
        m = jnp.max(mv[...])
        sm[0] = m + jnp.log(jnp.sum(sv[...] * jnp.exp(mv[...] - m)))

    @pl.when(p == 2)
    def _phase2():
        out_ref[...] = logits_s[i] - sm[0]


def _tc_call(counts2, tT, wT, b2):
    last = NBLK - 1
    return pl.pallas_call(
        _tc_body,
        grid=(3, NBLK),
        in_specs=[
            pl.BlockSpec((2, BV), lambda p, i: (0, jnp.where(p == 0, i, last))),
            pl.BlockSpec((EMB, BV), lambda p, i: (0, jnp.where(p == 0, i, last))),
            pl.BlockSpec((EMB, BV), lambda p, i: (0, jnp.where(p == 1, i, jnp.where(p == 0, 0, last)))),
            pl.BlockSpec((1, BV), lambda p, i: (0, jnp.where(p == 1, i, 0))),
        ],
        out_specs=pl.BlockSpec((1, BV), lambda p, i: (0, jnp.where(p == 2, i, 0))),
        out_shape=jax.ShapeDtypeStruct((1, VOCAB), jnp.float32),
        scratch_shapes=[
            pltpu.VMEM((1, EMB), jnp.float32),
            pltpu.VMEM((1, BV), jnp.float32),
            pltpu.VMEM((1, BV), jnp.float32),
            pltpu.VMEM((NBLK, 1, BV), jnp.float32),
            pltpu.SMEM((2,), jnp.float32),
        ],
    )(counts2, tT, wT, b2)


def kernel(indices, emb_table, W, b):
    idx = indices.astype(jnp.int32)
    counts_flat = _hist()(idx)
    counts2 = counts_flat.reshape(2, VOCAB_PAD)
    return _tc_call(counts2, emb_table.T, W.T, b.reshape(1, VOCAB))


# speedup vs baseline: 3.8295x; 1.1891x over previous
import functools

import jax
import jax.numpy as jnp
from jax import lax
from jax.experimental import pallas as pl
from jax.experimental.pallas import tpu as pltpu
from jax.experimental.pallas import tpu_sc as plsc

VOCAB = 100000
EMB = 64
N_CTX = 16384

NW = 32
PER_W = N_CTX // NW
CHUNK = 128
NCHUNK = PER_W // CHUNK

VOCAB_PAD = 100352
STRIPE = VOCAB_PAD // 16

BV = 14336
NBLK = VOCAB_PAD // BV


def _hist_body(idx_hbm, out_hbm, idx_v, zbuf, ones_v, shared, sem):
    core = lax.axis_index("c")
    sub = lax.axis_index("s")
    wid = sub * 2 + core

    z16 = jnp.zeros((16,), jnp.float32)

    def zero_body(j, _):
        zbuf[pl.ds(j * 16, 16)] = z16
        return 0

    lax.fori_loop(0, STRIPE // 16, zero_body, 0)
    for j in range(CHUNK // 16):
        ones_v[pl.ds(j * 16, 16)] = z16 + 1.0
    for k in range(NCHUNK):
        pltpu.sync_copy(
            idx_hbm.at[pl.ds(wid * PER_W + k * CHUNK, CHUNK)], idx_v.at[k]
        )
    pltpu.sync_copy(zbuf, shared.at[pl.ds(sub * STRIPE, STRIPE)])
    plsc.subcore_barrier()
    for k in range(NCHUNK):
        pltpu.sync_copy(ones_v, shared.at[idx_v.at[k]], add=True)
    plsc.subcore_barrier()
    pltpu.sync_copy(
        shared.at[pl.ds(sub * STRIPE, STRIPE)],
        out_hbm.at[pl.ds(core * VOCAB_PAD + sub * STRIPE, STRIPE)],
    )


@functools.cache
def _hist():
    return functools.partial(
        pl.kernel,
        out_type=jax.ShapeDtypeStruct((2 * VOCAB_PAD,), jnp.float32),
        mesh=plsc.VectorSubcoreMesh(core_axis_name="c", subcore_axis_name="s"),
        scratch_types=[
            pltpu.VMEM((NCHUNK, CHUNK), jnp.int32),
            pltpu.VMEM((STRIPE,), jnp.float32),
            pltpu.VMEM((CHUNK,), jnp.float32),
            pltpu.VMEM_SHARED((VOCAB_PAD,), jnp.float32),
            pltpu.SemaphoreType.DMA,
        ],
    )(_hist_body)


def _tc_body(counts_ref, t_ref, w_ref, b_ref, out_ref, s_acc, mv, sv, logits_s, sm):
    p = pl.program_id(0)
    i = pl.program_id(1)
    lane = lax.broadcasted_iota(jnp.int32, (1, BV), 1)
    nvalid = VOCAB - i * BV

    @pl.when((p == 0) & (i == 0))
    def _init_s():
        s_acc[...] = jnp.zeros((1, EMB), jnp.float32)

    @pl.when((p == 0) & (i < NBLK - 1))
    def _phase0():
        c2 = counts_ref[...]
        c = c2[0:1, :] + c2[1:2, :]
        s_acc[...] += lax.dot_general(
            c, t_ref[...], (((1,), (1,)), ((), ())),
            preferred_element_type=jnp.float32,
        )

    @pl.when((p == 0) & (i == NBLK - 1))
    def _phase0_edge():
        c2 = counts_ref[...]
        c = c2[0:1, :] + c2[1:2, :]
        t = jnp.where(lane < nvalid, t_ref[...], 0.0)
        s_acc[...] += lax.dot_general(
            c, t, (((1,), (1,)), ((), ())),
            preferred_element_type=jnp.float32,
        )

    @pl.when((p == 1) & (i == 0))
    def _init_ms():
        mv[...] = jnp.full((1, BV), -1e30, jnp.float32)
        sv[...] = jnp.zeros((1, BV), jnp.float32)

    @pl.when(p == 1)
    def _phase1():
        s2 = s_acc[...] * (1.0 / N_CTX)
        l = lax.dot_general(
            s2, w_ref[...], (((1,), (0,)), ((), ())),
            preferred_element_type=jnp.float32,
        ) + b_ref[...]
        l = jnp.where(lane < nvalid, l, -1e30)
        logits_s[i] = l
        m_old = mv[...]
        m_new = jnp.maximum(m_old, l)
        sv[...] = sv[...] * jnp.exp(m_old - m_new) + jnp.exp(l - m_new)
        mv[...] = m_new

    @pl.when((p == 2) & (i == 0))
    def _logz():
        m = jnp.max(mv[...])
        sm[0] = m + jnp.log(jnp.sum(sv[...] * jnp.exp(mv[...] - m)))

    @pl.when(p == 2)
    def _phase2():
        out_ref[...] = logits_s[i] - sm[0]


def _tc_call(counts2, tT, wT, b2):
    last = NBLK - 1
    return pl.pallas_call(
        _tc_body,
        grid=(3, NBLK),
        in_specs=[
            pl.BlockSpec((2, BV), lambda p, i: (0, jnp.where(p == 0, i, last))),
            pl.BlockSpec((EMB, BV), lambda p, i: (0, jnp.where(p == 0, i, last))),
            pl.BlockSpec((EMB, BV), lambda p, i: (0, jnp.where(p == 1, i, jnp.where(p == 0, 0, last)))),
            pl.BlockSpec((1, BV), lambda p, i: (0, jnp.where(p == 1, i, 0))),
        ],
        out_specs=pl.BlockSpec((1, BV), lambda p, i: (0, jnp.where(p == 2, i, 0))),
        out_shape=jax.ShapeDtypeStruct((1, VOCAB), jnp.float32),
        scratch_shapes=[
            pltpu.VMEM((1, EMB), jnp.float32),
            pltpu.VMEM((1, BV), jnp.float32),
            pltpu.VMEM((1, BV), jnp.float32),
            pltpu.VMEM((NBLK, 1, BV), jnp.float32),
            pltpu.SMEM((2,), jnp.float32),
        ],
    )(counts2, tT, wT, b2)


def kernel(indices, emb_table, W, b):
    idx = indices.astype(jnp.int32)
    counts_flat = _hist()(idx)
    counts2 = counts_flat.reshape(2, VOCAB_PAD)
    return _tc_call(counts2, emb_table.T, W.T, b.reshape(1, VOCAB))


# speedup vs baseline: 4.0599x; 1.0602x over previous
import functools

import jax
import jax.numpy as jnp
from jax import lax
from jax.experimental import pallas as pl
from jax.experimental.pallas import tpu as pltpu
from jax.experimental.pallas import tpu_sc as plsc

VOCAB = 100000
EMB = 64
N_CTX = 16384

NW = 32
PER_W = N_CTX // NW
CHUNK = 128
NCHUNK = PER_W // CHUNK

VOCAB_PAD = 100352
STRIPE = VOCAB_PAD // 16

BV = 25088
NBLK = VOCAB_PAD // BV


def _hist_body(idx_hbm, out_hbm, idx_v, zbuf, ones_v, shared, sem):
    core = lax.axis_index("c")
    sub = lax.axis_index("s")
    wid = sub * 2 + core

    z16 = jnp.zeros((16,), jnp.float32)

    def zero_body(j, _):
        zbuf[pl.ds(j * 16, 16)] = z16
        return 0

    lax.fori_loop(0, STRIPE // 16, zero_body, 0)
    for j in range(CHUNK // 16):
        ones_v[pl.ds(j * 16, 16)] = z16 + 1.0
    for k in range(NCHUNK):
        pltpu.sync_copy(
            idx_hbm.at[pl.ds(wid * PER_W + k * CHUNK, CHUNK)], idx_v.at[k]
        )
    pltpu.sync_copy(zbuf, shared.at[pl.ds(sub * STRIPE, STRIPE)])
    plsc.subcore_barrier()
    for k in range(NCHUNK):
        pltpu.sync_copy(ones_v, shared.at[idx_v.at[k]], add=True)
    plsc.subcore_barrier()
    pltpu.sync_copy(
        shared.at[pl.ds(sub * STRIPE, STRIPE)],
        out_hbm.at[pl.ds(core * VOCAB_PAD + sub * STRIPE, STRIPE)],
    )


@functools.cache
def _hist():
    return functools.partial(
        pl.kernel,
        out_type=jax.ShapeDtypeStruct((2 * VOCAB_PAD,), jnp.float32),
        mesh=plsc.VectorSubcoreMesh(core_axis_name="c", subcore_axis_name="s"),
        scratch_types=[
            pltpu.VMEM((NCHUNK, CHUNK), jnp.int32),
            pltpu.VMEM((STRIPE,), jnp.float32),
            pltpu.VMEM((CHUNK,), jnp.float32),
            pltpu.VMEM_SHARED((VOCAB_PAD,), jnp.float32),
            pltpu.SemaphoreType.DMA,
        ],
    )(_hist_body)


def _tc_body(counts_ref, t_ref, w_ref, b_ref, out_ref, s_acc, mv, sv, logits_s, sm):
    p = pl.program_id(0)
    i = pl.program_id(1)
    lane = lax.broadcasted_iota(jnp.int32, (1, BV), 1)
    nvalid = VOCAB - i * BV

    @pl.when((p == 0) & (i == 0))
    def _init_s():
        s_acc[...] = jnp.zeros((1, EMB), jnp.float32)

    @pl.when((p == 0) & (i < NBLK - 1))
    def _phase0():
        c2 = counts_ref[...]
        c = c2[0:1, :] + c2[1:2, :]
        s_acc[...] += lax.dot_general(
            c, t_ref[...], (((1,), (1,)), ((), ())),
            preferred_element_type=jnp.float32,
        )

    @pl.when((p == 0) & (i == NBLK - 1))
    def _phase0_edge():
        c2 = counts_ref[...]
        c = c2[0:1, :] + c2[1:2, :]
        t = jnp.where(lane < nvalid, t_ref[...], 0.0)
        s_acc[...] += lax.dot_general(
            c, t, (((1,), (1,)), ((), ())),
            preferred_element_type=jnp.float32,
        )

    @pl.when((p == 1) & (i == 0))
    def _init_ms():
        mv[...] = jnp.full((1, BV), -1e30, jnp.float32)
        sv[...] = jnp.zeros((1, BV), jnp.float32)

    @pl.when(p == 1)
    def _phase1():
        s2 = s_acc[...] * (1.0 / N_CTX)
        l = lax.dot_general(
            s2, w_ref[...], (((1,), (0,)), ((), ())),
            preferred_element_type=jnp.float32,
        ) + b_ref[...]
        l = jnp.where(lane < nvalid, l, -1e30)
        logits_s[i] = l
        m_old = mv[...]
        m_new = jnp.maximum(m_old, l)
        sv[...] = sv[...] * jnp.exp(m_old - m_new) + jnp.exp(l - m_new)
        mv[...] = m_new

    @pl.when((p == 2) & (i == 0))
    def _logz():
        m = jnp.max(mv[...])
        sm[0] = m + jnp.log(jnp.sum(sv[...] * jnp.exp(mv[...] - m)))

    @pl.when(p == 2)
    def _phase2():
        out_ref[...] = logits_s[i] - sm[0]


def _tc_call(counts2, tT, wT, b2):
    last = NBLK - 1
    return pl.pallas_call(
        _tc_body,
        grid=(3, NBLK),
        in_specs=[
            pl.BlockSpec((2, BV), lambda p, i: (0, jnp.where(p == 0, i, last))),
            pl.BlockSpec((EMB, BV), lambda p, i: (0, jnp.where(p == 0, i, last))),
            pl.BlockSpec((EMB, BV), lambda p, i: (0, jnp.where(p == 1, i, jnp.where(p == 0, 0, last)))),
            pl.BlockSpec((1, BV), lambda p, i: (0, jnp.where(p == 1, i, 0))),
        ],
        out_specs=pl.BlockSpec((1, BV), lambda p, i: (0, jnp.where(p == 2, i, 0))),
        out_shape=jax.ShapeDtypeStruct((1, VOCAB), jnp.float32),
        compiler_params=pltpu.CompilerParams(vmem_limit_bytes=100 * 1024 * 1024),
        scratch_shapes=[
            pltpu.VMEM((1, EMB), jnp.float32),
            pltpu.VMEM((1, BV), jnp.float32),
            pltpu.VMEM((1, BV), jnp.float32),
            pltpu.VMEM((NBLK, 1, BV), jnp.float32),
            pltpu.SMEM((2,), jnp.float32),
        ],
    )(counts2, tT, wT, b2)


def kernel(indices, emb_table, W, b):
    idx = indices.astype(jnp.int32)
    counts_flat = _hist()(idx)
    counts2 = counts_flat.reshape(2, VOCAB_PAD)
    return _tc_call(counts2, emb_table.T, W.T, b.reshape(1, VOCAB))


# speedup vs baseline: 4.6236x; 1.1388x over previous
import functools

import jax
import jax.numpy as jnp
from jax import lax
from jax.experimental import pallas as pl
from jax.experimental.pallas import tpu as pltpu
from jax.experimental.pallas import tpu_sc as plsc

VOCAB = 100000
EMB = 64
N_CTX = 16384

NW = 32
PER_W = N_CTX // NW
CHUNK = 128
NCHUNK = PER_W // CHUNK

VOCAB_PAD = 100352
STRIPE = VOCAB_PAD // 16

BV = 25088
NBLK = VOCAB_PAD // BV


def _hist_body(idx_hbm, out0_hbm, out1_hbm, idx_v, zbuf, ones_v, shared, sem):
    core = lax.axis_index("c")
    sub = lax.axis_index("s")
    wid = sub * 2 + core

    idx_copies = [
        pltpu.async_copy(
            idx_hbm.at[pl.ds(wid * PER_W + k * CHUNK, CHUNK)], idx_v.at[k], sem
        )
        for k in range(NCHUNK)
    ]

    z16 = jnp.zeros((16,), jnp.float32)

    def zero_body(j, _):
        for u in range(8):
            zbuf[pl.ds(j * 128 + u * 16, 16)] = z16
        return 0

    lax.fori_loop(0, STRIPE // 128, zero_body, 0)
    for j in range(CHUNK // 16):
        ones_v[pl.ds(j * 16, 16)] = z16 + 1.0
    pltpu.sync_copy(zbuf, shared.at[pl.ds(sub * STRIPE, STRIPE)])
    for c in idx_copies:
        c.wait()
    plsc.subcore_barrier()
    for k in range(NCHUNK):
        pltpu.sync_copy(ones_v, shared.at[idx_v.at[k]], add=True)
    plsc.subcore_barrier()

    @pl.when(core == 0)
    def _out0():
        pltpu.sync_copy(
            shared.at[pl.ds(sub * STRIPE, STRIPE)],
            out0_hbm.at[pl.ds(sub * STRIPE, STRIPE)],
        )

    @pl.when(core == 1)
    def _out1():
        pltpu.sync_copy(
            shared.at[pl.ds(sub * STRIPE, STRIPE)],
            out1_hbm.at[pl.ds(sub * STRIPE, STRIPE)],
        )


@functools.cache
def _hist():
    return functools.partial(
        pl.kernel,
        out_type=[
            jax.ShapeDtypeStruct((VOCAB_PAD,), jnp.float32),
            jax.ShapeDtypeStruct((VOCAB_PAD,), jnp.float32),
        ],
        mesh=plsc.VectorSubcoreMesh(core_axis_name="c", subcore_axis_name="s"),
        scratch_types=[
            pltpu.VMEM((NCHUNK, CHUNK), jnp.int32),
            pltpu.VMEM((STRIPE,), jnp.float32),
            pltpu.VMEM((CHUNK,), jnp.float32),
            pltpu.VMEM_SHARED((VOCAB_PAD,), jnp.float32),
            pltpu.SemaphoreType.DMA,
        ],
    )(_hist_body)


def _tc_body(c0_ref, c1_ref, t_ref, w_ref, b_ref, out_ref, s_acc, mv, sv, logits_s, sm):
    p = pl.program_id(0)
    i = pl.program_id(1)
    lane = lax.broadcasted_iota(jnp.int32, (1, BV), 1)
    nvalid = VOCAB - i * BV

    @pl.when((p == 0) & (i == 0))
    def _init_s():
        s_acc[...] = jnp.zeros((1, EMB), jnp.float32)

    @pl.when((p == 0) & (i < NBLK - 1))
    def _phase0():
        c = c0_ref[...] + c1_ref[...]
        s_acc[...] += lax.dot_general(
            c, t_ref[...], (((1,), (1,)), ((), ())),
            preferred_element_type=jnp.float32,
        )

    @pl.when((p == 0) & (i == NBLK - 1))
    def _phase0_edge():
        c = c0_ref[...] + c1_ref[...]
        t = jnp.where(lane < nvalid, t_ref[...], 0.0)
        s_acc[...] += lax.dot_general(
            c, t, (((1,), (1,)), ((), ())),
            preferred_element_type=jnp.float32,
        )

    @pl.when((p == 1) & (i == 0))
    def _init_ms():
        mv[...] = jnp.full((1, BV), -1e30, jnp.float32)
        sv[...] = jnp.zeros((1, BV), jnp.float32)

    @pl.when(p == 1)
    def _phase1():
        s2 = s_acc[...] * (1.0 / N_CTX)
        l = lax.dot_general(
            s2, w_ref[...], (((1,), (0,)), ((), ())),
            preferred_element_type=jnp.float32,
        ) + b_ref[...]
        l = jnp.where(lane < nvalid, l, -1e30)
        logits_s[i] = l
        m_old = mv[...]
        m_new = jnp.maximum(m_old, l)
        sv[...] = sv[...] * jnp.exp(m_old - m_new) + jnp.exp(l - m_new)
        mv[...] = m_new

    @pl.when((p == 2) & (i == 0))
    def _logz():
        m = jnp.max(mv[...])
        sm[0] = m + jnp.log(jnp.sum(sv[...] * jnp.exp(mv[...] - m)))

    @pl.when(p == 2)
    def _phase2():
        out_ref[...] = logits_s[i] - sm[0]


def _tc_call(c0, c1, tT, wT, b2):
    last = NBLK - 1
    return pl.pallas_call(
        _tc_body,
        grid=(3, NBLK),
        in_specs=[
            pl.BlockSpec((1, BV), lambda p, i: (0, jnp.where(p == 0, i, last))),
            pl.BlockSpec((1, BV), lambda p, i: (0, jnp.where(p == 0, i, last))),
            pl.BlockSpec((EMB, BV), lambda p, i: (0, jnp.where(p == 0, i, last))),
            pl.BlockSpec((EMB, BV), lambda p, i: (0, jnp.where(p == 1, i, jnp.where(p == 0, 0, last)))),
            pl.BlockSpec((1, BV), lambda p, i: (0, jnp.where(p == 1, i, 0))),
        ],
        out_specs=pl.BlockSpec((1, BV), lambda p, i: (0, jnp.where(p == 2, i, 0))),
        out_shape=jax.ShapeDtypeStruct((1, VOCAB), jnp.float32),
        compiler_params=pltpu.CompilerParams(vmem_limit_bytes=100 * 1024 * 1024),
        scratch_shapes=[
            pltpu.VMEM((1, EMB), jnp.float32),
            pltpu.VMEM((1, BV), jnp.float32),
            pltpu.VMEM((1, BV), jnp.float32),
            pltpu.VMEM((NBLK, 1, BV), jnp.float32),
            pltpu.SMEM((2,), jnp.float32),
        ],
    )(c0, c1, tT, wT, b2)


def kernel(indices, emb_table, W, b):
    idx = indices.astype(jnp.int32)
    c0, c1 = _hist()(idx)
    return _tc_call(
        c0.reshape(1, VOCAB_PAD),
        c1.reshape(1, VOCAB_PAD),
        emb_table.T,
        W.T,
        b.reshape(1, VOCAB),
    )
